# baseline (speedup 1.0000x reference)
import dataclasses

import jax
import jax.numpy as jnp
from jax import lax
from jax.experimental import pallas as pl
from jax.experimental.pallas import tpu as pltpu
from jax.experimental.pallas import tpu_sc as plsc

DIM = 128
LANES = 16
CHUNK = 64


def _make_sc_kernel(batch, num_nodes):
    info = plsc.get_sparse_core_info()
    nc, ns = info.num_cores, info.num_subcores
    nw = nc * ns
    bpw = batch // nw
    nchunk = bpw // CHUNK
    assert bpw * nw == batch and nchunk * CHUNK == bpw

    mesh = plsc.VectorSubcoreMesh(core_axis_name="c", subcore_axis_name="s")

    def body(src_idx_hbm, dst_idx_hbm, in_emb_hbm, out_emb_hbm, out_hbm,
             idx_s, idx_d, srcb, dstb, outv, sem):
        wid = lax.axis_index("s") * nc + lax.axis_index("c")
        base = wid * bpw
        ci = pltpu.async_copy(src_idx_hbm.at[pl.ds(base, bpw)], idx_s,
                              sem.at[0])
        cd = pltpu.async_copy(dst_idx_hbm.at[pl.ds(base, bpw)], idx_d,
                              sem.at[1])
        ci.wait()
        cd.wait()

        def start(c, slot):
            pltpu.async_copy(
                in_emb_hbm.at[idx_s.at[pl.ds(c * CHUNK, CHUNK)]],
                srcb.at[slot], sem.at[slot])
            pltpu.async_copy(
                out_emb_hbm.at[idx_d.at[pl.ds(c * CHUNK, CHUNK)]],
                dstb.at[slot], sem.at[slot])

        lane = lax.iota(jnp.int32, LANES)
        nslice = DIM // LANES
        bitrev8 = [0, 4, 2, 6, 1, 5, 3, 7]

        shuf_dnums = lax.GatherDimensionNumbers(
            offset_dims=(), collapsed_slice_dims=(0,), start_index_map=(0,))

        def shuf_idx(v, idxv):
            return lax.gather(v, idxv[:, None], shuf_dnums,
                              slice_sizes=(1,),
                              mode=lax.GatherScatterMode.PROMISE_IN_BOUNDS)

        def shuf(v, k):
            return shuf_idx(v, lane ^ k)

        def merge(a, b, k):
            m = (lane & k) == 0
            j1 = jnp.where(m, a, shuf(b, k))
            j2 = jnp.where(m, shuf(a, k), b)
            return j1 + j2

        even_lane = (lane & 1) == 0
        quad_lane = (lane & 3) == 0
        oct_lane = (lane & 7) == 0

        def row_partial(sref, dref, r):
            prods = [sref[r, pl.ds(k * LANES, LANES)]
                     * dref[r, pl.ds(k * LANES, LANES)]
                     for k in range(nslice)]
            while len(prods) > 1:
                prods = [prods[i] + prods[i + 1]
                         for i in range(0, len(prods), 2)]
            return prods[0]

        def tree8(sref, dref, r0):
            stack = []
            for idx in range(8):
                v, k, t = row_partial(sref, dref, r0 + bitrev8[idx]), 8, idx
                while t & 1:
                    v = merge(stack.pop(), v, k)
                    k >>= 1
                    t >>= 1
                stack.append(v)
            v = stack[0]
            return v + shuf(v, 1)

        def tree4(sref, dref, r0):
            a = merge(row_partial(sref, dref, r0),
                      row_partial(sref, dref, r0 + 2), 8)
            b = merge(row_partial(sref, dref, r0 + 1),
                      row_partial(sref, dref, r0 + 3), 8)
            v = merge(a, b, 4)
            v = v + shuf(v, 2)
            return v + shuf(v, 1)

        def compute_chunk(slot, c):
            sref = srcb.at[slot]
            dref = dstb.at[slot]

            @pl.loop(0, CHUNK, step=4)
            def _(r0):
                va = tree4(sref, dref, r0)
                plsc.store_compressed(outv.at[pl.ds(c * CHUNK + r0, LANES)],
                                      va, mask=quad_lane)

        start(0, 0)
        start(1, 1)

        @pl.loop(0, nchunk)
        def _(c):
            s = c & 3
            @pl.when(c + 2 < nchunk)
            def _():
                start(c + 2, (c + 2) & 3)

            pltpu.make_async_copy(in_emb_hbm.at[pl.ds(0, CHUNK)],
                                  srcb.at[s], sem.at[s]).wait()
            pltpu.make_async_copy(out_emb_hbm.at[pl.ds(0, CHUNK)],
                                  dstb.at[s], sem.at[s]).wait()

            compute_chunk(s, c)

        pltpu.sync_copy(outv.at[pl.ds(0, bpw)], out_hbm.at[pl.ds(base, bpw)])

    cp = pltpu.CompilerParams()
    if "needs_layout_passes" in pltpu.CompilerParams.__dataclass_fields__:
        cp = dataclasses.replace(cp, needs_layout_passes=False)

    return pl.kernel(
        body,
        out_type=jax.ShapeDtypeStruct((batch,), jnp.float32),
        mesh=mesh,
        compiler_params=cp,
        scratch_types=[
            pltpu.VMEM((bpw,), jnp.int32),
            pltpu.VMEM((bpw,), jnp.int32),
            pltpu.VMEM((4, CHUNK, DIM), jnp.float32),
            pltpu.VMEM((4, CHUNK, DIM), jnp.float32),
            pltpu.VMEM((bpw + LANES,), jnp.float32),
            pltpu.SemaphoreType.DMA((4,)),
        ],
    )


def kernel(src_idx, dst_idx, in_emb, out_emb):
    batch = src_idx.shape[0]
    num_nodes = in_emb.shape[0]
    sc = _make_sc_kernel(batch, num_nodes)
    return sc(src_idx.astype(jnp.int32), dst_idx.astype(jnp.int32),
              in_emb, out_emb)


# speedup vs baseline: 1.0310x; 1.0310x over previous
import dataclasses

import jax
import jax.numpy as jnp
from jax import lax
from jax.experimental import pallas as pl
from jax.experimental.pallas import tpu as pltpu
from jax.experimental.pallas import tpu_sc as plsc

DIM = 128
LANES = 16
CHUNK = 64


def _make_sc_kernel(batch, num_nodes):
    info = plsc.get_sparse_core_info()
    nc, ns = info.num_cores, info.num_subcores
    nw = nc * ns
    bpw = batch // nw
    nchunk = bpw // CHUNK
    assert bpw * nw == batch and nchunk * CHUNK == bpw

    mesh = plsc.VectorSubcoreMesh(core_axis_name="c", subcore_axis_name="s")

    def body(src_idx_hbm, dst_idx_hbm, in_emb_hbm, out_emb_hbm, out_hbm,
             idx_s, idx_d, srcb, dstb, outv, sem):
        wid = lax.axis_index("s") * nc + lax.axis_index("c")
        base = wid * bpw
        ci = pltpu.async_copy(src_idx_hbm.at[pl.ds(base, bpw)], idx_s,
                              sem.at[0])
        cd = pltpu.async_copy(dst_idx_hbm.at[pl.ds(base, bpw)], idx_d,
                              sem.at[1])
        ci.wait()
        cd.wait()

        def start(c, slot):
            pltpu.async_copy(
                in_emb_hbm.at[idx_s.at[pl.ds(c * CHUNK, CHUNK)]],
                srcb.at[slot], sem.at[slot])
            pltpu.async_copy(
                out_emb_hbm.at[idx_d.at[pl.ds(c * CHUNK, CHUNK)]],
                dstb.at[slot], sem.at[slot])

        lane = lax.iota(jnp.int32, LANES)
        nslice = DIM // LANES
        bitrev8 = [0, 4, 2, 6, 1, 5, 3, 7]

        shuf_dnums = lax.GatherDimensionNumbers(
            offset_dims=(), collapsed_slice_dims=(0,), start_index_map=(0,))

        def shuf_idx(v, idxv):
            return lax.gather(v, idxv[:, None], shuf_dnums,
                              slice_sizes=(1,),
                              mode=lax.GatherScatterMode.PROMISE_IN_BOUNDS)

        def shuf(v, k):
            return shuf_idx(v, lane ^ k)

        def merge(a, b, k):
            m = (lane & k) == 0
            j1 = jnp.where(m, a, shuf(b, k))
            j2 = jnp.where(m, shuf(a, k), b)
            return j1 + j2

        even_lane = (lane & 1) == 0
        quad_lane = (lane & 3) == 0
        oct_lane = (lane & 7) == 0

        def row_partial(sref, dref, r):
            prods = [sref[r, pl.ds(k * LANES, LANES)]
                     * dref[r, pl.ds(k * LANES, LANES)]
                     for k in range(nslice)]
            while len(prods) > 1:
                prods = [prods[i] + prods[i + 1]
                         for i in range(0, len(prods), 2)]
            return prods[0]

        def tree8(sref, dref, r0):
            stack = []
            for idx in range(8):
                v, k, t = row_partial(sref, dref, r0 + bitrev8[idx]), 8, idx
                while t & 1:
                    v = merge(stack.pop(), v, k)
                    k >>= 1
                    t >>= 1
                stack.append(v)
            v = stack[0]
            return v + shuf(v, 1)

        def tree4(sref, dref, r0):
            a = merge(row_partial(sref, dref, r0),
                      row_partial(sref, dref, r0 + 2), 8)
            b = merge(row_partial(sref, dref, r0 + 1),
                      row_partial(sref, dref, r0 + 3), 8)
            v = merge(a, b, 4)
            v = v + shuf(v, 2)
            return v + shuf(v, 1)

        def compute_chunk(slot, c):
            sref = srcb.at[slot]
            dref = dstb.at[slot]

            @pl.loop(0, CHUNK, step=4)
            def _(r0):
                va = tree4(sref, dref, r0)
                plsc.store_compressed(outv.at[pl.ds(c * CHUNK + r0, LANES)],
                                      va, mask=quad_lane)

        start(0, 0)
        start(1, 1)

        @pl.loop(0, nchunk)
        def _(c):
            s = c & 3
            pltpu.make_async_copy(in_emb_hbm.at[pl.ds(0, CHUNK)],
                                  srcb.at[s], sem.at[s]).wait()
            pltpu.make_async_copy(out_emb_hbm.at[pl.ds(0, CHUNK)],
                                  dstb.at[s], sem.at[s]).wait()

            @pl.when(c + 2 < nchunk)
            def _():
                start(c + 2, (c + 2) & 3)

            compute_chunk(s, c)

        pltpu.sync_copy(outv.at[pl.ds(0, bpw)], out_hbm.at[pl.ds(base, bpw)])

    cp = pltpu.CompilerParams()
    if "needs_layout_passes" in pltpu.CompilerParams.__dataclass_fields__:
        cp = dataclasses.replace(cp, needs_layout_passes=False)

    return pl.kernel(
        body,
        out_type=jax.ShapeDtypeStruct((batch,), jnp.float32),
        mesh=mesh,
        compiler_params=cp,
        scratch_types=[
            pltpu.VMEM((bpw,), jnp.int32),
            pltpu.VMEM((bpw,), jnp.int32),
            pltpu.VMEM((4, CHUNK, DIM), jnp.float32),
            pltpu.VMEM((4, CHUNK, DIM), jnp.float32),
            pltpu.VMEM((bpw + LANES,), jnp.float32),
            pltpu.SemaphoreType.DMA((4,)),
        ],
    )


def kernel(src_idx, dst_idx, in_emb, out_emb):
    batch = src_idx.shape[0]
    num_nodes = in_emb.shape[0]
    sc = _make_sc_kernel(batch, num_nodes)
    return sc(src_idx.astype(jnp.int32), dst_idx.astype(jnp.int32),
              in_emb, out_emb)


# speedup vs baseline: 1.0321x; 1.0011x over previous
import dataclasses

import jax
import jax.numpy as jnp
from jax import lax
from jax.experimental import pallas as pl
from jax.experimental.pallas import tpu as pltpu
from jax.experimental.pallas import tpu_sc as plsc

DIM = 128
LANES = 16
CHUNK = 64


def _make_sc_kernel(batch, num_nodes):
    info = plsc.get_sparse_core_info()
    nc, ns = info.num_cores, info.num_subcores
    nw = nc * ns
    bpw = batch // nw
    nchunk = bpw // CHUNK
    assert bpw * nw == batch and nchunk * CHUNK == bpw

    mesh = plsc.VectorSubcoreMesh(core_axis_name="c", subcore_axis_name="s")

    def body(src_idx_hbm, dst_idx_hbm, in_emb_hbm, out_emb_hbm, out_hbm,
             idx_s, idx_d, srcb, dstb, outv, sem):
        wid = lax.axis_index("s") * nc + lax.axis_index("c")
        base = wid * bpw
        ci = pltpu.async_copy(src_idx_hbm.at[pl.ds(base, bpw)], idx_s,
                              sem.at[0])
        cd = pltpu.async_copy(dst_idx_hbm.at[pl.ds(base, bpw)], idx_d,
                              sem.at[1])
        ci.wait()
        cd.wait()

        def start(c, slot):
            pltpu.async_copy(
                in_emb_hbm.at[idx_s.at[pl.ds(c * CHUNK, CHUNK)]],
                srcb.at[slot], sem.at[slot])
            pltpu.async_copy(
                out_emb_hbm.at[idx_d.at[pl.ds(c * CHUNK, CHUNK)]],
                dstb.at[slot], sem.at[slot])

        lane = lax.iota(jnp.int32, LANES)
        nslice = DIM // LANES
        bitrev8 = [0, 4, 2, 6, 1, 5, 3, 7]

        shuf_dnums = lax.GatherDimensionNumbers(
            offset_dims=(), collapsed_slice_dims=(0,), start_index_map=(0,))

        def shuf_idx(v, idxv):
            return lax.gather(v, idxv[:, None], shuf_dnums,
                              slice_sizes=(1,),
                              mode=lax.GatherScatterMode.PROMISE_IN_BOUNDS)

        def shuf(v, k):
            return shuf_idx(v, lane ^ k)

        def merge(a, b, k):
            m = (lane & k) == 0
            j1 = jnp.where(m, a, shuf(b, k))
            j2 = jnp.where(m, shuf(a, k), b)
            return j1 + j2

        even_lane = (lane & 1) == 0
        quad_lane = (lane & 3) == 0
        oct_lane = (lane & 7) == 0

        def row_partial(sref, dref, r):
            prods = [sref[r, pl.ds(k * LANES, LANES)]
                     * dref[r, pl.ds(k * LANES, LANES)]
                     for k in range(nslice)]
            while len(prods) > 1:
                prods = [prods[i] + prods[i + 1]
                         for i in range(0, len(prods), 2)]
            return prods[0]

        def tree8(sref, dref, r0):
            stack = []
            for idx in range(8):
                v, k, t = row_partial(sref, dref, r0 + bitrev8[idx]), 8, idx
                while t & 1:
                    v = merge(stack.pop(), v, k)
                    k >>= 1
                    t >>= 1
                stack.append(v)
            v = stack[0]
            return v + shuf(v, 1)

        def tree4(sref, dref, r0):
            a = merge(row_partial(sref, dref, r0),
                      row_partial(sref, dref, r0 + 2), 8)
            b = merge(row_partial(sref, dref, r0 + 1),
                      row_partial(sref, dref, r0 + 3), 8)
            v = merge(a, b, 4)
            v = v + shuf(v, 2)
            return v + shuf(v, 1)

        def compute_chunk(slot, c):
            sref = srcb.at[slot]
            dref = dstb.at[slot]

            @pl.loop(0, CHUNK, step=4)
            def _(r0):
                va = tree4(sref, dref, r0)
                plsc.store_compressed(outv.at[pl.ds(c * CHUNK + r0, LANES)],
                                      va, mask=quad_lane)

        start(0, 0)
        start(1, 1)

        @pl.loop(0, nchunk)
        def _(c):
            s = c & 3
            pltpu.make_async_copy(in_emb_hbm.at[pl.ds(0, CHUNK)],
                                  srcb.at[s], sem.at[s]).wait()
            pltpu.make_async_copy(out_emb_hbm.at[pl.ds(0, CHUNK)],
                                  dstb.at[s], sem.at[s]).wait()

            @pl.when(c + 2 < nchunk)
            def _():
                start(c + 2, (c + 2) & 3)

            compute_chunk(s, c)
            pltpu.async_copy(outv.at[pl.ds(c * CHUNK, CHUNK)],
                             out_hbm.at[pl.ds(base + c * CHUNK, CHUNK)],
                             sem.at[4])

        @pl.loop(0, nchunk)
        def _(c):
            pltpu.make_async_copy(outv.at[pl.ds(0, CHUNK)],
                                  out_hbm.at[pl.ds(base, CHUNK)],
                                  sem.at[4]).wait()

    cp = pltpu.CompilerParams()
    if "needs_layout_passes" in pltpu.CompilerParams.__dataclass_fields__:
        cp = dataclasses.replace(cp, needs_layout_passes=False)

    return pl.kernel(
        body,
        out_type=jax.ShapeDtypeStruct((batch,), jnp.float32),
        mesh=mesh,
        compiler_params=cp,
        scratch_types=[
            pltpu.VMEM((bpw,), jnp.int32),
            pltpu.VMEM((bpw,), jnp.int32),
            pltpu.VMEM((4, CHUNK, DIM), jnp.float32),
            pltpu.VMEM((4, CHUNK, DIM), jnp.float32),
            pltpu.VMEM((bpw + LANES,), jnp.float32),
            pltpu.SemaphoreType.DMA((5,)),
        ],
    )


def kernel(src_idx, dst_idx, in_emb, out_emb):
    batch = src_idx.shape[0]
    num_nodes = in_emb.shape[0]
    sc = _make_sc_kernel(batch, num_nodes)
    return sc(src_idx.astype(jnp.int32), dst_idx.astype(jnp.int32),
              in_emb, out_emb)


# speedup vs baseline: 1.0331x; 1.0009x over previous
import dataclasses

import jax
import jax.numpy as jnp
from jax import lax
from jax.experimental import pallas as pl
from jax.experimental.pallas import tpu as pltpu
from jax.experimental.pallas import tpu_sc as plsc

DIM = 128
LANES = 16
CHUNK = 64


def _make_sc_kernel(batch, num_nodes):
    info = plsc.get_sparse_core_info()
    nc, ns = info.num_cores, info.num_subcores
    nw = nc * ns
    bpw = batch // nw
    nchunk = bpw // CHUNK
    assert bpw * nw == batch and nchunk * CHUNK == bpw

    mesh = plsc.VectorSubcoreMesh(core_axis_name="c", subcore_axis_name="s")

    def body(src_idx_hbm, dst_idx_hbm, in_emb_hbm, out_emb_hbm, out_hbm,
             idx_s, idx_d, srcb, dstb, outv, sem):
        wid = lax.axis_index("s") * nc + lax.axis_index("c")
        base = wid * bpw
        ci = pltpu.async_copy(src_idx_hbm.at[pl.ds(base, bpw)], idx_s,
                              sem.at[0])
        cd = pltpu.async_copy(dst_idx_hbm.at[pl.ds(base, bpw)], idx_d,
                              sem.at[1])
        ci.wait()
        cd.wait()

        def start(c, slot):
            pltpu.async_copy(
                in_emb_hbm.at[idx_s.at[pl.ds(c * CHUNK, CHUNK)]],
                srcb.at[slot], sem.at[slot])
            pltpu.async_copy(
                out_emb_hbm.at[idx_d.at[pl.ds(c * CHUNK, CHUNK)]],
                dstb.at[slot], sem.at[slot])

        lane = lax.iota(jnp.int32, LANES)
        nslice = DIM // LANES
        bitrev8 = [0, 4, 2, 6, 1, 5, 3, 7]

        shuf_dnums = lax.GatherDimensionNumbers(
            offset_dims=(), collapsed_slice_dims=(0,), start_index_map=(0,))

        def shuf_idx(v, idxv):
            return lax.gather(v, idxv[:, None], shuf_dnums,
                              slice_sizes=(1,),
                              mode=lax.GatherScatterMode.PROMISE_IN_BOUNDS)

        def shuf(v, k):
            return shuf_idx(v, lane ^ k)

        def merge(a, b, k):
            m = (lane & k) == 0
            j1 = jnp.where(m, a, shuf(b, k))
            j2 = jnp.where(m, shuf(a, k), b)
            return j1 + j2

        even_lane = (lane & 1) == 0
        quad_lane = (lane & 3) == 0
        oct_lane = (lane & 7) == 0

        def row_partial(sref, dref, r):
            prods = [sref[r, pl.ds(k * LANES, LANES)]
                     * dref[r, pl.ds(k * LANES, LANES)]
                     for k in range(nslice)]
            while len(prods) > 1:
                prods = [prods[i] + prods[i + 1]
                         for i in range(0, len(prods), 2)]
            return prods[0]

        def tree8(sref, dref, r0):
            stack = []
            for idx in range(8):
                v, k, t = row_partial(sref, dref, r0 + bitrev8[idx]), 8, idx
                while t & 1:
                    v = merge(stack.pop(), v, k)
                    k >>= 1
                    t >>= 1
                stack.append(v)
            v = stack[0]
            return v + shuf(v, 1)

        def tree4(sref, dref, r0):
            a = merge(row_partial(sref, dref, r0),
                      row_partial(sref, dref, r0 + 2), 8)
            b = merge(row_partial(sref, dref, r0 + 1),
                      row_partial(sref, dref, r0 + 3), 8)
            v = merge(a, b, 4)
            v = v + shuf(v, 2)
            return v + shuf(v, 1)

        def compute_chunk(slot, c):
            sref = srcb.at[slot]
            dref = dstb.at[slot]

            @pl.loop(0, CHUNK, step=4)
            def _(r0):
                va = tree4(sref, dref, r0)
                plsc.store_compressed(outv.at[pl.ds(c * CHUNK + r0, LANES)],
                                      va, mask=quad_lane)

        start(0, 0)
        start(1, 1)

        @pl.loop(0, nchunk)
        def _(c):
            s = c & 3
            pltpu.make_async_copy(in_emb_hbm.at[pl.ds(0, CHUNK)],
                                  srcb.at[s], sem.at[s]).wait()
            pltpu.make_async_copy(out_emb_hbm.at[pl.ds(0, CHUNK)],
                                  dstb.at[s], sem.at[s]).wait()

            @pl.when(c + 2 < nchunk)
            def _():
                start(c + 2, (c + 2) & 3)

            compute_chunk(s, c)

        pltpu.sync_copy(outv.at[pl.ds(0, bpw)], out_hbm.at[pl.ds(base, bpw)])

    cp = pltpu.CompilerParams()
    if "needs_layout_passes" in pltpu.CompilerParams.__dataclass_fields__:
        cp = dataclasses.replace(cp, needs_layout_passes=False)

    return pl.kernel(
        body,
        out_type=jax.ShapeDtypeStruct((batch,), jnp.float32),
        mesh=mesh,
        compiler_params=cp,
        scratch_types=[
            pltpu.VMEM((bpw,), jnp.int32),
            pltpu.VMEM((bpw,), jnp.int32),
            pltpu.VMEM((4, CHUNK, DIM), jnp.float32),
            pltpu.VMEM((4, CHUNK, DIM), jnp.float32),
            pltpu.VMEM((bpw + LANES,), jnp.float32),
            pltpu.SemaphoreType.DMA((4,)),
        ],
    )


def kernel(src_idx, dst_idx, in_emb, out_emb):
    batch = src_idx.shape[0]
    num_nodes = in_emb.shape[0]
    sc = _make_sc_kernel(batch, num_nodes)
    return sc(src_idx.astype(jnp.int32), dst_idx.astype(jnp.int32),
              in_emb, out_emb)
